# baseline (speedup 1.0000x reference)
import jax
import jax.numpy as jnp
from jax.experimental import pallas as pl
from jax.experimental.pallas import tpu as pltpu

TSTEPS = 2000
NB = 256
H = 224
HH = H // 2
IMG4 = (NB, 1, H, H)
BR = 8


def _body(t_sm, gam_sm, s1_sm, ya, yb, na, nb, o_ref):
    i = pl.program_id(0)
    for r in range(BR):
        idx = t_sm[i * BR + r]
        g = gam_sm[idx]
        s = s1_sm[idx]
        o_ref[r, 0, :HH] = g * ya[r, 0] + s * na[r, 0]
        o_ref[r, 0, HH:] = g * yb[r, 0] + s * nb[r, 0]


_half = lambda j: pl.BlockSpec((BR, 1, HH, H), lambda i, t, g, s, _j=j: (i, 0, _j, 0))

_scale_add_call = pl.pallas_call(
    _body,
    grid_spec=pltpu.PrefetchScalarGridSpec(
        num_scalar_prefetch=3,
        grid=(NB // BR,),
        in_specs=[_half(0), _half(1), _half(0), _half(1)],
        out_specs=[
            pl.BlockSpec((BR, 1, H, H), lambda i, t, g, s: (i, 0, 0, 0)),
        ],
    ),
    out_shape=[
        jax.ShapeDtypeStruct(IMG4, jnp.float32),
    ],
)


def kernel(y, noise, t, gammas, sqrt_one_minus_gammas, sqrt_gammas):
    t32 = t.astype(jnp.int32)
    (oy,) = _scale_add_call(t32, gammas, sqrt_one_minus_gammas, y, y, noise, noise)
    return oy, noise


# speedup vs baseline: 3.5959x; 3.5959x over previous
import jax
import jax.numpy as jnp
from jax import lax
from jax.experimental import pallas as pl
from jax.experimental.pallas import tpu as pltpu

TSTEPS = 2000
TPAD = 2048
NB = 256
H = 224
ROWS = H * H
BS = 1792


def _body(t_ref, tbl_ref, y_ref, n_ref, oy_ref, on_ref, coef_ref):
    @pl.when(pl.program_id(0) == 0)
    def _():
        k = lax.broadcasted_iota(jnp.int32, (TPAD, NB), 0)
        hit = k == t_ref[...]
        for j in range(2):
            contrib = jnp.where(hit, tbl_ref[:, j:j + 1], 0.0)
            coef_ref[j:j + 1, :] = jnp.sum(contrib, axis=0, keepdims=True)

    g = coef_ref[0:1, :]
    s = coef_ref[1:2, :]
    nv = n_ref[...]
    oy_ref[...] = g * y_ref[...] + s * nv
    on_ref[...] = nv


_scale_add_call = pl.pallas_call(
    _body,
    grid=(ROWS // BS,),
    in_specs=[
        pl.BlockSpec((1, NB), lambda i: (0, 0)),
        pl.BlockSpec((TPAD, 2), lambda i: (0, 0)),
        pl.BlockSpec((BS, NB), lambda i: (i, 0)),
        pl.BlockSpec((BS, NB), lambda i: (i, 0)),
    ],
    out_specs=[
        pl.BlockSpec((BS, NB), lambda i: (i, 0)),
        pl.BlockSpec((BS, NB), lambda i: (i, 0)),
    ],
    out_shape=[
        jax.ShapeDtypeStruct((ROWS, NB), jnp.float32),
        jax.ShapeDtypeStruct((ROWS, NB), jnp.float32),
    ],
    scratch_shapes=[pltpu.VMEM((2, NB), jnp.float32)],
)


def kernel(y, noise, t, gammas, sqrt_one_minus_gammas, sqrt_gammas):
    t2 = t.astype(jnp.int32).reshape(1, NB)
    tbl2 = jnp.stack(
        [jnp.pad(gammas, (0, TPAD - TSTEPS)),
         jnp.pad(sqrt_one_minus_gammas, (0, TPAD - TSTEPS))],
        axis=1,
    )
    y2 = y.transpose(1, 2, 3, 0).reshape(ROWS, NB)
    n2 = noise.transpose(1, 2, 3, 0).reshape(ROWS, NB)
    oy2, on2 = _scale_add_call(t2, tbl2, y2, n2)
    oy = oy2.reshape(1, H, H, NB).transpose(3, 0, 1, 2)
    on = on2.reshape(1, H, H, NB).transpose(3, 0, 1, 2)
    return oy, on


# speedup vs baseline: 3.6539x; 1.0162x over previous
import jax
import jax.numpy as jnp
from jax import lax
from jax.experimental import pallas as pl
from jax.experimental.pallas import tpu as pltpu

TSTEPS = 2000
TPAD = 2048
NB = 256
H = 224
ROWS = H * H
BS = 3584


def _body(t_ref, tbl_ref, y_ref, n_ref, oy_ref, on_ref, coef_ref):
    @pl.when(pl.program_id(0) == 0)
    def _():
        k = lax.broadcasted_iota(jnp.int32, (TPAD, NB), 0)
        hit = k == t_ref[...]
        for j in range(2):
            contrib = jnp.where(hit, tbl_ref[:, j:j + 1], 0.0)
            coef_ref[j:j + 1, :] = jnp.sum(contrib, axis=0, keepdims=True)

    g = coef_ref[0:1, :]
    s = coef_ref[1:2, :]
    nv = n_ref[...]
    oy_ref[...] = g * y_ref[...] + s * nv
    on_ref[...] = nv


_scale_add_call = pl.pallas_call(
    _body,
    grid=(ROWS // BS,),
    in_specs=[
        pl.BlockSpec((1, NB), lambda i: (0, 0)),
        pl.BlockSpec((TPAD, 2), lambda i: (0, 0)),
        pl.BlockSpec((BS, NB), lambda i: (i, 0)),
        pl.BlockSpec((BS, NB), lambda i: (i, 0)),
    ],
    out_specs=[
        pl.BlockSpec((BS, NB), lambda i: (i, 0)),
        pl.BlockSpec((BS, NB), lambda i: (i, 0)),
    ],
    out_shape=[
        jax.ShapeDtypeStruct((ROWS, NB), jnp.float32),
        jax.ShapeDtypeStruct((ROWS, NB), jnp.float32),
    ],
    scratch_shapes=[pltpu.VMEM((2, NB), jnp.float32)],
)


def kernel(y, noise, t, gammas, sqrt_one_minus_gammas, sqrt_gammas):
    t2 = t.astype(jnp.int32).reshape(1, NB)
    tbl2 = jnp.stack(
        [jnp.pad(gammas, (0, TPAD - TSTEPS)),
         jnp.pad(sqrt_one_minus_gammas, (0, TPAD - TSTEPS))],
        axis=1,
    )
    y2 = y.transpose(1, 2, 3, 0).reshape(ROWS, NB)
    n2 = noise.transpose(1, 2, 3, 0).reshape(ROWS, NB)
    oy2, on2 = _scale_add_call(t2, tbl2, y2, n2)
    oy = oy2.reshape(1, H, H, NB).transpose(3, 0, 1, 2)
    on = on2.reshape(1, H, H, NB).transpose(3, 0, 1, 2)
    return oy, on


# speedup vs baseline: 3.7808x; 1.0347x over previous
import jax
import jax.numpy as jnp
from jax import lax
from jax.experimental import pallas as pl
from jax.experimental.pallas import tpu as pltpu

TSTEPS = 2000
TPAD = 2048
NB = 256
H = 224
ROWS = H * H
BS = 7168


def _body(t_ref, tbl_ref, y_ref, n_ref, oy_ref, on_ref, coef_ref):
    @pl.when(pl.program_id(0) == 0)
    def _():
        k = lax.broadcasted_iota(jnp.int32, (TPAD, NB), 0)
        hit = k == t_ref[...]
        for j in range(2):
            contrib = jnp.where(hit, tbl_ref[:, j:j + 1], 0.0)
            coef_ref[j:j + 1, :] = jnp.sum(contrib, axis=0, keepdims=True)

    g = coef_ref[0:1, :]
    s = coef_ref[1:2, :]
    nv = n_ref[...]
    oy_ref[...] = g * y_ref[...] + s * nv
    on_ref[...] = nv


_scale_add_call = pl.pallas_call(
    _body,
    grid=(ROWS // BS,),
    in_specs=[
        pl.BlockSpec((1, NB), lambda i: (0, 0)),
        pl.BlockSpec((TPAD, 2), lambda i: (0, 0)),
        pl.BlockSpec((BS, NB), lambda i: (i, 0)),
        pl.BlockSpec((BS, NB), lambda i: (i, 0)),
    ],
    out_specs=[
        pl.BlockSpec((BS, NB), lambda i: (i, 0)),
        pl.BlockSpec((BS, NB), lambda i: (i, 0)),
    ],
    out_shape=[
        jax.ShapeDtypeStruct((ROWS, NB), jnp.float32),
        jax.ShapeDtypeStruct((ROWS, NB), jnp.float32),
    ],
    scratch_shapes=[pltpu.VMEM((2, NB), jnp.float32)],
)


def kernel(y, noise, t, gammas, sqrt_one_minus_gammas, sqrt_gammas):
    t2 = t.astype(jnp.int32).reshape(1, NB)
    tbl2 = jnp.stack(
        [jnp.pad(gammas, (0, TPAD - TSTEPS)),
         jnp.pad(sqrt_one_minus_gammas, (0, TPAD - TSTEPS))],
        axis=1,
    )
    y2 = y.transpose(1, 2, 3, 0).reshape(ROWS, NB)
    n2 = noise.transpose(1, 2, 3, 0).reshape(ROWS, NB)
    oy2, on2 = _scale_add_call(t2, tbl2, y2, n2)
    oy = oy2.reshape(1, H, H, NB).transpose(3, 0, 1, 2)
    on = on2.reshape(1, H, H, NB).transpose(3, 0, 1, 2)
    return oy, on


# speedup vs baseline: 3.9561x; 1.0464x over previous
import jax
import jax.numpy as jnp
from jax import lax
from jax.experimental import pallas as pl
from jax.experimental.pallas import tpu as pltpu

NB = 256
H = 224
ROWS = H * H
BS = 7168


def _body(t_sm, gam_sm, s1_sm, y_ref, n_ref, oy_ref, on_ref,
          coef_vmem, coef_smem, sem):
    @pl.when(pl.program_id(0) == 0)
    def _():
        def lp(b, carry):
            idx = t_sm[b]
            coef_smem[0, b] = gam_sm[idx]
            coef_smem[1, b] = s1_sm[idx]
            return carry

        lax.fori_loop(0, NB, lp, 0)
        cp = pltpu.make_async_copy(coef_smem, coef_vmem, sem)
        cp.start()
        cp.wait()

    g = coef_vmem[0:1, :]
    s = coef_vmem[1:2, :]
    nv = n_ref[...]
    oy_ref[...] = g * y_ref[...] + s * nv
    on_ref[...] = nv


_scale_add_call = pl.pallas_call(
    _body,
    grid_spec=pltpu.PrefetchScalarGridSpec(
        num_scalar_prefetch=3,
        grid=(ROWS // BS,),
        in_specs=[
            pl.BlockSpec((BS, NB), lambda i, t, g, s: (i, 0)),
            pl.BlockSpec((BS, NB), lambda i, t, g, s: (i, 0)),
        ],
        out_specs=[
            pl.BlockSpec((BS, NB), lambda i, t, g, s: (i, 0)),
            pl.BlockSpec((BS, NB), lambda i, t, g, s: (i, 0)),
        ],
        scratch_shapes=[
            pltpu.VMEM((2, NB), jnp.float32),
            pltpu.SMEM((2, NB), jnp.float32),
            pltpu.SemaphoreType.DMA,
        ],
    ),
    out_shape=[
        jax.ShapeDtypeStruct((ROWS, NB), jnp.float32),
        jax.ShapeDtypeStruct((ROWS, NB), jnp.float32),
    ],
)


def kernel(y, noise, t, gammas, sqrt_one_minus_gammas, sqrt_gammas):
    t32 = t.astype(jnp.int32)
    y2 = y.transpose(1, 2, 3, 0).reshape(ROWS, NB)
    n2 = noise.transpose(1, 2, 3, 0).reshape(ROWS, NB)
    oy2, on2 = _scale_add_call(t32, gammas, sqrt_one_minus_gammas, y2, n2)
    oy = oy2.reshape(1, H, H, NB).transpose(3, 0, 1, 2)
    on = on2.reshape(1, H, H, NB).transpose(3, 0, 1, 2)
    return oy, on
